# Initial kernel scaffold; baseline (speedup 1.0000x reference)
import jax
import jax.numpy as jnp
from jax.experimental import pallas as pl


def kernel(x_nchw, w_proj, w_cls, b_cls):
    raise NotImplementedError("write your pallas kernel here")


# speedup vs baseline: 1.0377x; 1.0377x over previous
import functools

import jax
import jax.numpy as jnp
from jax.experimental import pallas as pl
from jax.experimental.pallas import tpu as pltpu


def _wcomb_body(wp_ref, wc_ref, out_ref, *, inv_s):
    out_ref[...] = jnp.dot(
        wp_ref[...], wc_ref[...], preferred_element_type=jnp.float32
    ) * inv_s


def _pool_head_body(x_ref, w_ref, b_ref, out_ref, *, TB, C, NCLS):
    TR = TB * C
    pooled = jnp.sum(x_ref[...].astype(jnp.float32), axis=-1, keepdims=True)
    w_full = jnp.broadcast_to(w_ref[...][None], (TB, C, NCLS)).reshape(TR, NCLS)
    scaled = w_full * pooled
    r = jax.lax.broadcasted_iota(jnp.int32, (TB, TR), 1)
    b = jax.lax.broadcasted_iota(jnp.int32, (TB, TR), 0)
    sel = jnp.where((r // C) == b, 1.0, 0.0)
    logits = jnp.dot(sel, scaled, preferred_element_type=jnp.float32)
    out_ref[...] = logits + b_ref[...]


def kernel(x_nchw, w_proj, w_cls, b_cls):
    B, C, H, W = x_nchw.shape
    S = H * W
    NCLS = w_cls.shape[1]

    w_comb = pl.pallas_call(
        functools.partial(_wcomb_body, inv_s=1.0 / float(S)),
        out_shape=jax.ShapeDtypeStruct((C, NCLS), jnp.float32),
    )(w_proj.astype(jnp.float32), w_cls.astype(jnp.float32))

    TB = 8 if B % 8 == 0 else 1
    NR = B // TB
    TR = TB * C

    x2d = x_nchw.reshape(B * C, S)
    bias = b_cls.astype(jnp.float32).reshape(1, NCLS)

    out = pl.pallas_call(
        functools.partial(_pool_head_body, TB=TB, C=C, NCLS=NCLS),
        out_shape=jax.ShapeDtypeStruct((B, NCLS), jnp.float32),
        grid=(NR,),
        in_specs=[
            pl.BlockSpec((TR, S), lambda i: (i, 0)),
            pl.BlockSpec((C, NCLS), lambda i: (0, 0)),
            pl.BlockSpec((1, NCLS), lambda i: (0, 0)),
        ],
        out_specs=pl.BlockSpec((TB, NCLS), lambda i: (i, 0)),
        compiler_params=pltpu.CompilerParams(
            dimension_semantics=("parallel",),
            vmem_limit_bytes=48 << 20,
        ),
    )(x2d, w_comb, bias)
    return out
